# baseline (speedup 1.0000x reference)
import jax
import jax.numpy as jnp
from jax import lax
from jax.experimental import pallas as pl
from jax.experimental.pallas import tpu as pltpu
from jax.experimental.pallas import tpu_sc as plsc

N_NODES = 10000
D_FEAT = 128
N_EDGES = 320000

NUM_CORES = 2
NUM_SUBCORES = 16
LANES = 128
ROWS_PER_TILE = 80
E_PAD = NUM_CORES * NUM_SUBCORES * ROWS_PER_TILE * LANES
IDX_ROWS = E_PAD // LANES

ACC_ROWS = 10240
ZCHUNK = 128
ROWS_PER_TILE_ZERO = ACC_ROWS // NUM_SUBCORES
OUT_ROWS_PER_TILE = 632


PASSES = 2
PASS_ROWS = ROWS_PER_TILE // PASSES


def _sc_layer_body(xc_hbm, xa_hbm, xb_hbm,
                   sIa, dIa, sIb, dIb, sI1, dI1, sI2, dI2,
                   out_a, out_b, out_1, out_2,
                   src_idx, dst_idx, rows, gsem, ssem0, ssem1, acc):
    cid = lax.axis_index("c")
    sid = lax.axis_index("s")

    z16 = jnp.zeros((16,), jnp.float32)

    zbase = sid * ROWS_PER_TILE_ZERO
    base_row = (cid * NUM_SUBCORES + sid) * ROWS_PER_TILE
    ob = jnp.minimum(sid * OUT_ROWS_PER_TILE, N_NODES - OUT_ROWS_PER_TILE)

    convs = ((xc_hbm, sIa, dIa, out_a), (xc_hbm, sIb, dIb, out_b),
             (xa_hbm, sI1, dI1, out_1), (xb_hbm, sI2, dI2, out_2))

    for xsrc_hbm, srcI_hbm, dstI_hbm, sum_out in convs:
        def fill_z(i, carry):
            r = i // 8
            col = (i % 8) * 16
            rows[0, r, pl.ds(col, 16)] = z16
            return carry
        lax.fori_loop(0, LANES * 8, fill_z, 0)

        def zero_chunk(k, carry):
            pltpu.sync_copy(rows.at[0],
                            acc.at[pl.ds(zbase + k * ZCHUNK, ZCHUNK)])
            return carry
        lax.fori_loop(0, ROWS_PER_TILE_ZERO // ZCHUNK, zero_chunk, 0)

        plsc.subcore_barrier()

        for p in range(PASSES):
            pbase = base_row + p * PASS_ROWS
            pltpu.sync_copy(srcI_hbm.at[pl.ds(pbase, PASS_ROWS)], src_idx)
            pltpu.sync_copy(dstI_hbm.at[pl.ds(pbase, PASS_ROWS)], dst_idx)
            pltpu.async_copy(xsrc_hbm.at[src_idx.at[0]], rows.at[0], gsem)

            def step(j, carry):
                b = j & 1
                pltpu.make_async_copy(
                    xsrc_hbm.at[pl.ds(0, LANES)], rows.at[b], gsem).wait()

                @pl.when(j < PASS_ROWS - 1)
                def _():
                    pltpu.async_copy(
                        xsrc_hbm.at[src_idx.at[j + 1]], rows.at[1 - b], gsem)

                pltpu.sync_copy(rows.at[b], acc.at[dst_idx.at[j]], add=True)
                return carry
            lax.fori_loop(0, PASS_ROWS, step, 0)

        plsc.subcore_barrier()

        pltpu.sync_copy(acc.at[pl.ds(ob, OUT_ROWS_PER_TILE)],
                        sum_out.at[cid, pl.ds(ob, OUT_ROWS_PER_TILE)])
        plsc.subcore_barrier()


def _sc_layer(xc, xa, xb, e_c2a, e_c2b, e_a2c, e_b2c):
    fn = pl.kernel(
        _sc_layer_body,
        mesh=plsc.VectorSubcoreMesh(core_axis_name="c", subcore_axis_name="s"),
        out_type=[
            jax.ShapeDtypeStruct((NUM_CORES, N_NODES, D_FEAT), jnp.float32),
        ] * 4,
        scratch_types=[
            pltpu.VMEM((PASS_ROWS, LANES), jnp.int32),
            pltpu.VMEM((PASS_ROWS, LANES), jnp.int32),
            pltpu.VMEM((2, LANES, D_FEAT), jnp.float32),
            pltpu.SemaphoreType.DMA,
            pltpu.SemaphoreType.DMA,
            pltpu.SemaphoreType.DMA,
            pltpu.VMEM_SHARED((ACC_ROWS, D_FEAT), jnp.float32),
        ],
    )
    return fn(xc, xa, xb, e_c2a[0], e_c2a[1], e_c2b[0], e_c2b[1],
              e_a2c[0], e_a2c[1], e_b2c[0], e_b2c[1])


def _sc_deg_body(dI0, dI1, dI2, dI3, deg_out, dst_idx, ones_v, zdbuf, dacc):
    cid = lax.axis_index("c")
    sid = lax.axis_index("s")

    z16 = jnp.zeros((16,), jnp.float32)
    o16 = jnp.ones((16,), jnp.float32)

    def fill_zd(i, carry):
        zdbuf[pl.ds(i * 16, 16)] = z16
        return carry
    lax.fori_loop(0, ROWS_PER_TILE_ZERO // 16, fill_zd, 0)

    def fill_ones(i, carry):
        ones_v[pl.ds(i * 16, 16)] = o16
        return carry
    lax.fori_loop(0, LANES // 16, fill_ones, 0)

    zbase = sid * ROWS_PER_TILE_ZERO
    base_row = (cid * NUM_SUBCORES + sid) * ROWS_PER_TILE

    for et, dI in enumerate((dI0, dI1, dI2, dI3)):
        pltpu.sync_copy(zdbuf, dacc.at[pl.ds(zbase, ROWS_PER_TILE_ZERO)])
        plsc.subcore_barrier()
        pltpu.sync_copy(dI.at[pl.ds(base_row, ROWS_PER_TILE)], dst_idx)

        def step(j, carry):
            pltpu.sync_copy(ones_v, dacc.at[dst_idx.at[j]], add=True)
            return carry
        lax.fori_loop(0, ROWS_PER_TILE, step, 0)
        plsc.subcore_barrier()
        pltpu.sync_copy(
            dacc.at[pl.ds(zbase, ROWS_PER_TILE_ZERO)],
            deg_out.at[pl.ds(et * NUM_CORES * ACC_ROWS + cid * ACC_ROWS
                             + zbase, ROWS_PER_TILE_ZERO)])


def _sc_deg(d0, d1, d2, d3):
    fn = pl.kernel(
        _sc_deg_body,
        mesh=plsc.VectorSubcoreMesh(core_axis_name="c", subcore_axis_name="s"),
        out_type=[
            jax.ShapeDtypeStruct((4 * NUM_CORES * ACC_ROWS,), jnp.float32),
        ],
        scratch_types=[
            pltpu.VMEM((ROWS_PER_TILE, LANES), jnp.int32),
            pltpu.VMEM((LANES,), jnp.float32),
            pltpu.VMEM((ROWS_PER_TILE_ZERO,), jnp.float32),
            pltpu.VMEM_SHARED((ACC_ROWS,), jnp.float32),
        ],
    )
    return fn(d0, d1, d2, d3)[0]


ROW_BLK = 1024
N_BLK = 10


def _tc_layer_body(pa, d0a, d1a, pb, d0b, d1b, p1, d01, d11, p2, d02, d12,
                   xc, xa, xb,
                   wla, wra, ba, wlb, wrb, bb, wl1, wl2, wrc, bc,
                   oc, oa, ob_ref):
    def mean(p, d0, d1):
        s = p[0] + p[1]
        deg = jnp.maximum(d0[...] + d1[...], 1.0)
        return s / deg[:, None]

    def lrelu(x):
        return jnp.where(x > 0, x, 0.01 * x)

    m_a = mean(pa[...], d0a, d1a)
    out_a = (jnp.dot(m_a, wla[...], preferred_element_type=jnp.float32)
             + jnp.dot(xa[...], wra[...], preferred_element_type=jnp.float32)
             + ba[...])
    oa[...] = lrelu(out_a)

    m_b = mean(pb[...], d0b, d1b)
    out_b = (jnp.dot(m_b, wlb[...], preferred_element_type=jnp.float32)
             + jnp.dot(xb[...], wrb[...], preferred_element_type=jnp.float32)
             + bb[...])
    ob_ref[...] = lrelu(out_b)

    m_1 = mean(p1[...], d01, d11)
    m_2 = mean(p2[...], d02, d12)
    out_c = (jnp.dot(m_1, wl1[...], preferred_element_type=jnp.float32)
             + jnp.dot(m_2, wl2[...], preferred_element_type=jnp.float32)
             + jnp.dot(xc[...], wrc[...], preferred_element_type=jnp.float32)
             + bc[...])
    oc[...] = lrelu(out_c)


def _tc_layer(pa, da, pb, db, p1, d1, p2, d2, xc, xa, xb,
              wla, wra, ba, wlb, wrb, bb, wl1, wl2, wrc, bc):
    p_spec = pl.BlockSpec((NUM_CORES, ROW_BLK, D_FEAT), lambda i: (0, i, 0))
    d_spec = pl.BlockSpec((ROW_BLK,), lambda i: (i,))
    x_spec = pl.BlockSpec((ROW_BLK, D_FEAT), lambda i: (i, 0))
    w_spec = pl.BlockSpec((D_FEAT, D_FEAT), lambda i: (0, 0))
    b_spec = pl.BlockSpec((1, D_FEAT), lambda i: (0, 0))
    degs = [da, db, d1, d2]
    return pl.pallas_call(
        _tc_layer_body,
        grid=(N_BLK,),
        in_specs=[p_spec, d_spec, d_spec, p_spec, d_spec, d_spec,
                  p_spec, d_spec, d_spec, p_spec, d_spec, d_spec,
                  x_spec, x_spec, x_spec,
                  w_spec, w_spec, b_spec, w_spec, w_spec, b_spec,
                  w_spec, w_spec, w_spec, b_spec],
        out_specs=[x_spec, x_spec, x_spec],
        out_shape=[jax.ShapeDtypeStruct((N_NODES, D_FEAT), jnp.float32)] * 3,
    )(pa, *degs[0], pb, *degs[1], p1, *degs[2], p2, *degs[3],
      xc, xa, xb,
      wla, wra, ba, wlb, wrb, bb, wl1, wl2, wrc, bc)


def _prep_edges(ei):
    pad = E_PAD - N_EDGES
    src = jnp.concatenate(
        [ei[0].astype(jnp.int32), jnp.zeros((pad,), jnp.int32)])
    dst = jnp.concatenate(
        [ei[1].astype(jnp.int32), jnp.full((pad,), N_NODES, jnp.int32)])
    return src.reshape(IDX_ROWS, LANES), dst.reshape(IDX_ROWS, LANES)


def kernel(x_cdr3b, x_tra_peptide, x_trb_peptide, edge_index_c2a,
           edge_index_c2b, edge_index_a2c, edge_index_b2c, params):
    xc, xa, xb = x_cdr3b, x_tra_peptide, x_trb_peptide
    e_c2a = _prep_edges(edge_index_c2a)
    e_c2b = _prep_edges(edge_index_c2b)
    e_a2c = _prep_edges(edge_index_a2c)
    e_b2c = _prep_edges(edge_index_b2c)

    deg_all = _sc_deg(e_c2a[1], e_c2b[1], e_a2c[1], e_b2c[1])
    degs = []
    for et in range(4):
        base = et * NUM_CORES * ACC_ROWS
        degs.append((deg_all[base:base + ACC_ROWS],
                     deg_all[base + ACC_ROWS:base + 2 * ACC_ROWS]))

    for lp in params:
        wla, ba, wra = lp["c2a"]
        wlb, bb, wrb = lp["c2b"]
        wl1, b1, wr1 = lp["a2c"]
        wl2, b2, wr2 = lp["b2c"]
        wrc = wr1 + wr2
        bc = (b1 + b2).reshape(1, D_FEAT)

        pa, pb, p1, p2 = _sc_layer(xc, xa, xb, e_c2a, e_c2b, e_a2c, e_b2c)

        xc, xa, xb = _tc_layer(
            pa, degs[0], pb, degs[1], p1, degs[2], p2, degs[3], xc, xa, xb,
            wla, wra, ba.reshape(1, D_FEAT),
            wlb, wrb, bb.reshape(1, D_FEAT),
            wl1, wl2, wrc, bc)

    return (xc, xa, xb)


# speedup vs baseline: 1.0023x; 1.0023x over previous
import jax
import jax.numpy as jnp
from jax import lax
from jax.experimental import pallas as pl
from jax.experimental.pallas import tpu as pltpu
from jax.experimental.pallas import tpu_sc as plsc

N_NODES = 10000
D_FEAT = 128
N_EDGES = 320000

NUM_CORES = 2
NUM_SUBCORES = 16
LANES = 128
ROWS_PER_TILE = 80
E_PAD = NUM_CORES * NUM_SUBCORES * ROWS_PER_TILE * LANES
IDX_ROWS = E_PAD // LANES

ACC_ROWS = 10240
ZCHUNK = 128
ROWS_PER_TILE_ZERO = ACC_ROWS // NUM_SUBCORES
OUT_ROWS_PER_TILE = 632


PASSES = 2
PASS_ROWS = ROWS_PER_TILE // PASSES


def _sc_layer_body(xc_hbm, xa_hbm, xb_hbm,
                   sIa, dIa, sIb, dIb, sI1, dI1, sI2, dI2,
                   out_a, out_b, out_1, out_2,
                   src_idx, dst_idx, rows, gsem, ssem0, ssem1, acc):
    cid = lax.axis_index("c")
    sid = lax.axis_index("s")

    z16 = jnp.zeros((16,), jnp.float32)

    zbase = sid * ROWS_PER_TILE_ZERO
    base_row = (cid * NUM_SUBCORES + sid) * ROWS_PER_TILE
    ob = jnp.minimum(sid * OUT_ROWS_PER_TILE, N_NODES - OUT_ROWS_PER_TILE)

    convs = ((xc_hbm, sIa, dIa, out_a), (xc_hbm, sIb, dIb, out_b),
             (xa_hbm, sI1, dI1, out_1), (xb_hbm, sI2, dI2, out_2))

    for xsrc_hbm, srcI_hbm, dstI_hbm, sum_out in convs:
        def fill_z(i, carry):
            r = i // 8
            col = (i % 8) * 16
            rows[0, r, pl.ds(col, 16)] = z16
            return carry
        lax.fori_loop(0, LANES * 8, fill_z, 0)

        def zero_chunk(k, carry):
            pltpu.sync_copy(rows.at[0],
                            acc.at[pl.ds(zbase + k * ZCHUNK, ZCHUNK)])
            return carry
        lax.fori_loop(0, ROWS_PER_TILE_ZERO // ZCHUNK, zero_chunk, 0)

        plsc.subcore_barrier()

        for p in range(PASSES):
            pbase = base_row + p * PASS_ROWS
            pltpu.sync_copy(srcI_hbm.at[pl.ds(pbase, PASS_ROWS)], src_idx)
            pltpu.sync_copy(dstI_hbm.at[pl.ds(pbase, PASS_ROWS)], dst_idx)
            pltpu.async_copy(xsrc_hbm.at[src_idx.at[0]], rows.at[0], gsem)

            def step(j, carry):
                b = j & 1
                pltpu.make_async_copy(
                    xsrc_hbm.at[pl.ds(0, LANES)], rows.at[b], gsem).wait()

                @pl.when(b == 0)
                def _():
                    pltpu.async_copy(rows.at[0], acc.at[dst_idx.at[j]],
                                     ssem0, add=True)

                @pl.when(b == 1)
                def _():
                    pltpu.async_copy(rows.at[1], acc.at[dst_idx.at[j]],
                                     ssem1, add=True)

                @pl.when(j >= 1)
                def _():
                    @pl.when(b == 0)
                    def _():
                        pltpu.make_async_copy(
                            xsrc_hbm.at[pl.ds(0, LANES)], rows.at[1],
                            ssem1).wait()

                    @pl.when(b == 1)
                    def _():
                        pltpu.make_async_copy(
                            xsrc_hbm.at[pl.ds(0, LANES)], rows.at[0],
                            ssem0).wait()

                @pl.when(j < PASS_ROWS - 1)
                def _():
                    pltpu.async_copy(
                        xsrc_hbm.at[src_idx.at[j + 1]], rows.at[1 - b], gsem)
                return carry
            lax.fori_loop(0, PASS_ROWS, step, 0)
            pltpu.make_async_copy(
                xsrc_hbm.at[pl.ds(0, LANES)],
                rows.at[(PASS_ROWS - 1) & 1],
                ssem1 if (PASS_ROWS - 1) & 1 else ssem0).wait()

        plsc.subcore_barrier()

        pltpu.sync_copy(acc.at[pl.ds(ob, OUT_ROWS_PER_TILE)],
                        sum_out.at[cid, pl.ds(ob, OUT_ROWS_PER_TILE)])
        plsc.subcore_barrier()


def _sc_layer(xc, xa, xb, e_c2a, e_c2b, e_a2c, e_b2c):
    fn = pl.kernel(
        _sc_layer_body,
        mesh=plsc.VectorSubcoreMesh(core_axis_name="c", subcore_axis_name="s"),
        out_type=[
            jax.ShapeDtypeStruct((NUM_CORES, N_NODES, D_FEAT), jnp.float32),
        ] * 4,
        scratch_types=[
            pltpu.VMEM((PASS_ROWS, LANES), jnp.int32),
            pltpu.VMEM((PASS_ROWS, LANES), jnp.int32),
            pltpu.VMEM((2, LANES, D_FEAT), jnp.float32),
            pltpu.SemaphoreType.DMA,
            pltpu.SemaphoreType.DMA,
            pltpu.SemaphoreType.DMA,
            pltpu.VMEM_SHARED((ACC_ROWS, D_FEAT), jnp.float32),
        ],
    )
    return fn(xc, xa, xb, e_c2a[0], e_c2a[1], e_c2b[0], e_c2b[1],
              e_a2c[0], e_a2c[1], e_b2c[0], e_b2c[1])


def _sc_deg_body(dI0, dI1, dI2, dI3, deg_out, dst_idx, ones_v, zdbuf, dacc):
    cid = lax.axis_index("c")
    sid = lax.axis_index("s")

    z16 = jnp.zeros((16,), jnp.float32)
    o16 = jnp.ones((16,), jnp.float32)

    def fill_zd(i, carry):
        zdbuf[pl.ds(i * 16, 16)] = z16
        return carry
    lax.fori_loop(0, ROWS_PER_TILE_ZERO // 16, fill_zd, 0)

    def fill_ones(i, carry):
        ones_v[pl.ds(i * 16, 16)] = o16
        return carry
    lax.fori_loop(0, LANES // 16, fill_ones, 0)

    zbase = sid * ROWS_PER_TILE_ZERO
    base_row = (cid * NUM_SUBCORES + sid) * ROWS_PER_TILE

    for et, dI in enumerate((dI0, dI1, dI2, dI3)):
        pltpu.sync_copy(zdbuf, dacc.at[pl.ds(zbase, ROWS_PER_TILE_ZERO)])
        plsc.subcore_barrier()
        pltpu.sync_copy(dI.at[pl.ds(base_row, ROWS_PER_TILE)], dst_idx)

        def step(j, carry):
            pltpu.sync_copy(ones_v, dacc.at[dst_idx.at[j]], add=True)
            return carry
        lax.fori_loop(0, ROWS_PER_TILE, step, 0)
        plsc.subcore_barrier()
        pltpu.sync_copy(
            dacc.at[pl.ds(zbase, ROWS_PER_TILE_ZERO)],
            deg_out.at[pl.ds(et * NUM_CORES * ACC_ROWS + cid * ACC_ROWS
                             + zbase, ROWS_PER_TILE_ZERO)])


def _sc_deg(d0, d1, d2, d3):
    fn = pl.kernel(
        _sc_deg_body,
        mesh=plsc.VectorSubcoreMesh(core_axis_name="c", subcore_axis_name="s"),
        out_type=[
            jax.ShapeDtypeStruct((4 * NUM_CORES * ACC_ROWS,), jnp.float32),
        ],
        scratch_types=[
            pltpu.VMEM((ROWS_PER_TILE, LANES), jnp.int32),
            pltpu.VMEM((LANES,), jnp.float32),
            pltpu.VMEM((ROWS_PER_TILE_ZERO,), jnp.float32),
            pltpu.VMEM_SHARED((ACC_ROWS,), jnp.float32),
        ],
    )
    return fn(d0, d1, d2, d3)[0]


ROW_BLK = 1024
N_BLK = 10


def _tc_layer_body(pa, d0a, d1a, pb, d0b, d1b, p1, d01, d11, p2, d02, d12,
                   xc, xa, xb,
                   wla, wra, ba, wlb, wrb, bb, wl1, wl2, wrc, bc,
                   oc, oa, ob_ref):
    def mean(p, d0, d1):
        s = p[0] + p[1]
        deg = jnp.maximum(d0[...] + d1[...], 1.0)
        return s / deg[:, None]

    def lrelu(x):
        return jnp.where(x > 0, x, 0.01 * x)

    m_a = mean(pa[...], d0a, d1a)
    out_a = (jnp.dot(m_a, wla[...], preferred_element_type=jnp.float32)
             + jnp.dot(xa[...], wra[...], preferred_element_type=jnp.float32)
             + ba[...])
    oa[...] = lrelu(out_a)

    m_b = mean(pb[...], d0b, d1b)
    out_b = (jnp.dot(m_b, wlb[...], preferred_element_type=jnp.float32)
             + jnp.dot(xb[...], wrb[...], preferred_element_type=jnp.float32)
             + bb[...])
    ob_ref[...] = lrelu(out_b)

    m_1 = mean(p1[...], d01, d11)
    m_2 = mean(p2[...], d02, d12)
    out_c = (jnp.dot(m_1, wl1[...], preferred_element_type=jnp.float32)
             + jnp.dot(m_2, wl2[...], preferred_element_type=jnp.float32)
             + jnp.dot(xc[...], wrc[...], preferred_element_type=jnp.float32)
             + bc[...])
    oc[...] = lrelu(out_c)


def _tc_layer(pa, da, pb, db, p1, d1, p2, d2, xc, xa, xb,
              wla, wra, ba, wlb, wrb, bb, wl1, wl2, wrc, bc):
    p_spec = pl.BlockSpec((NUM_CORES, ROW_BLK, D_FEAT), lambda i: (0, i, 0))
    d_spec = pl.BlockSpec((ROW_BLK,), lambda i: (i,))
    x_spec = pl.BlockSpec((ROW_BLK, D_FEAT), lambda i: (i, 0))
    w_spec = pl.BlockSpec((D_FEAT, D_FEAT), lambda i: (0, 0))
    b_spec = pl.BlockSpec((1, D_FEAT), lambda i: (0, 0))
    degs = [da, db, d1, d2]
    return pl.pallas_call(
        _tc_layer_body,
        grid=(N_BLK,),
        in_specs=[p_spec, d_spec, d_spec, p_spec, d_spec, d_spec,
                  p_spec, d_spec, d_spec, p_spec, d_spec, d_spec,
                  x_spec, x_spec, x_spec,
                  w_spec, w_spec, b_spec, w_spec, w_spec, b_spec,
                  w_spec, w_spec, w_spec, b_spec],
        out_specs=[x_spec, x_spec, x_spec],
        out_shape=[jax.ShapeDtypeStruct((N_NODES, D_FEAT), jnp.float32)] * 3,
    )(pa, *degs[0], pb, *degs[1], p1, *degs[2], p2, *degs[3],
      xc, xa, xb,
      wla, wra, ba, wlb, wrb, bb, wl1, wl2, wrc, bc)


def _prep_edges(ei):
    pad = E_PAD - N_EDGES
    src = jnp.concatenate(
        [ei[0].astype(jnp.int32), jnp.zeros((pad,), jnp.int32)])
    dst = jnp.concatenate(
        [ei[1].astype(jnp.int32), jnp.full((pad,), N_NODES, jnp.int32)])
    return src.reshape(IDX_ROWS, LANES), dst.reshape(IDX_ROWS, LANES)


def kernel(x_cdr3b, x_tra_peptide, x_trb_peptide, edge_index_c2a,
           edge_index_c2b, edge_index_a2c, edge_index_b2c, params):
    xc, xa, xb = x_cdr3b, x_tra_peptide, x_trb_peptide
    e_c2a = _prep_edges(edge_index_c2a)
    e_c2b = _prep_edges(edge_index_c2b)
    e_a2c = _prep_edges(edge_index_a2c)
    e_b2c = _prep_edges(edge_index_b2c)

    deg_all = _sc_deg(e_c2a[1], e_c2b[1], e_a2c[1], e_b2c[1])
    degs = []
    for et in range(4):
        base = et * NUM_CORES * ACC_ROWS
        degs.append((deg_all[base:base + ACC_ROWS],
                     deg_all[base + ACC_ROWS:base + 2 * ACC_ROWS]))

    for lp in params:
        wla, ba, wra = lp["c2a"]
        wlb, bb, wrb = lp["c2b"]
        wl1, b1, wr1 = lp["a2c"]
        wl2, b2, wr2 = lp["b2c"]
        wrc = wr1 + wr2
        bc = (b1 + b2).reshape(1, D_FEAT)

        pa, pb, p1, p2 = _sc_layer(xc, xa, xb, e_c2a, e_c2b, e_a2c, e_b2c)

        xc, xa, xb = _tc_layer(
            pa, degs[0], pb, degs[1], p1, degs[2], p2, degs[3], xc, xa, xb,
            wla, wra, ba.reshape(1, D_FEAT),
            wlb, wrb, bb.reshape(1, D_FEAT),
            wl1, wl2, wrc, bc)

    return (xc, xa, xb)


# speedup vs baseline: 1.0393x; 1.0369x over previous
import jax
import jax.numpy as jnp
from jax import lax
from jax.experimental import pallas as pl
from jax.experimental.pallas import tpu as pltpu
from jax.experimental.pallas import tpu_sc as plsc

N_NODES = 10000
D_FEAT = 128
N_EDGES = 320000

NUM_CORES = 2
NUM_SUBCORES = 16
LANES = 128
CHUNK = 64
NBUF = 4
E_PAD = 327680
IDX_ROWS = E_PAD // CHUNK
CHUNKS_PER_TILE = IDX_ROWS // (NUM_CORES * NUM_SUBCORES)
ROWS_PER_TILE = 80

ACC_ROWS = 10240
ZCHUNK = 64
ROWS_PER_TILE_ZERO = ACC_ROWS // NUM_SUBCORES
OUT_ROWS_PER_TILE = 632


PASSES = 4
PASS_ROWS = CHUNKS_PER_TILE // PASSES


def _sc_layer_body(xc_hbm, xa_hbm, xb_hbm,
                   sIa, dIa, sIb, dIb, sI1, dI1, sI2, dI2,
                   out_a, out_b, out_1, out_2,
                   src_idx, dst_idx, rows, gsem, acc):
    cid = lax.axis_index("c")
    sid = lax.axis_index("s")

    z16 = jnp.zeros((16,), jnp.float32)

    zbase = sid * ROWS_PER_TILE_ZERO
    base_row = (cid * NUM_SUBCORES + sid) * CHUNKS_PER_TILE
    ob = jnp.minimum(sid * OUT_ROWS_PER_TILE, N_NODES - OUT_ROWS_PER_TILE)

    convs = ((xc_hbm, sIa, dIa, out_a), (xc_hbm, sIb, dIb, out_b),
             (xa_hbm, sI1, dI1, out_1), (xb_hbm, sI2, dI2, out_2))

    for xsrc_hbm, srcI_hbm, dstI_hbm, sum_out in convs:
        def fill_z(i, carry):
            r = i // 8
            col = (i % 8) * 16
            rows[0, r, pl.ds(col, 16)] = z16
            return carry
        lax.fori_loop(0, CHUNK * 8, fill_z, 0)

        def zero_chunk(k, carry):
            pltpu.sync_copy(rows.at[0],
                            acc.at[pl.ds(zbase + k * ZCHUNK, ZCHUNK)])
            return carry
        lax.fori_loop(0, ROWS_PER_TILE_ZERO // ZCHUNK, zero_chunk, 0)

        plsc.subcore_barrier()

        for p in range(PASSES):
            pbase = base_row + p * PASS_ROWS
            pltpu.sync_copy(srcI_hbm.at[pl.ds(pbase, PASS_ROWS)], src_idx)
            pltpu.sync_copy(dstI_hbm.at[pl.ds(pbase, PASS_ROWS)], dst_idx)
            for jj in range(NBUF - 1):
                pltpu.async_copy(
                    xsrc_hbm.at[src_idx.at[jj]], rows.at[jj], gsem)

            def step(j, carry):
                b = j & (NBUF - 1)
                pltpu.make_async_copy(
                    xsrc_hbm.at[pl.ds(0, CHUNK)], rows.at[b], gsem).wait()

                @pl.when(j + NBUF - 1 < PASS_ROWS)
                def _():
                    pltpu.async_copy(
                        xsrc_hbm.at[src_idx.at[j + NBUF - 1]],
                        rows.at[(j + NBUF - 1) & (NBUF - 1)], gsem)

                pltpu.sync_copy(rows.at[b], acc.at[dst_idx.at[j]], add=True)
                return carry
            lax.fori_loop(0, PASS_ROWS, step, 0)

        plsc.subcore_barrier()

        pltpu.sync_copy(acc.at[pl.ds(ob, OUT_ROWS_PER_TILE)],
                        sum_out.at[cid, pl.ds(ob, OUT_ROWS_PER_TILE)])
        plsc.subcore_barrier()


def _sc_layer(xc, xa, xb, e_c2a, e_c2b, e_a2c, e_b2c):
    fn = pl.kernel(
        _sc_layer_body,
        mesh=plsc.VectorSubcoreMesh(core_axis_name="c", subcore_axis_name="s"),
        out_type=[
            jax.ShapeDtypeStruct((NUM_CORES, N_NODES, D_FEAT), jnp.float32),
        ] * 4,
        scratch_types=[
            pltpu.VMEM((PASS_ROWS, CHUNK), jnp.int32),
            pltpu.VMEM((PASS_ROWS, CHUNK), jnp.int32),
            pltpu.VMEM((NBUF, CHUNK, D_FEAT), jnp.float32),
            pltpu.SemaphoreType.DMA,
            pltpu.VMEM_SHARED((ACC_ROWS, D_FEAT), jnp.float32),
        ],
    )
    return fn(xc, xa, xb, e_c2a[0], e_c2a[1], e_c2b[0], e_c2b[1],
              e_a2c[0], e_a2c[1], e_b2c[0], e_b2c[1])


def _sc_deg_body(dI0, dI1, dI2, dI3, deg_out, dst_idx, ones_v, zdbuf, dacc):
    cid = lax.axis_index("c")
    sid = lax.axis_index("s")

    z16 = jnp.zeros((16,), jnp.float32)
    o16 = jnp.ones((16,), jnp.float32)

    def fill_zd(i, carry):
        zdbuf[pl.ds(i * 16, 16)] = z16
        return carry
    lax.fori_loop(0, ROWS_PER_TILE_ZERO // 16, fill_zd, 0)

    def fill_ones(i, carry):
        ones_v[pl.ds(i * 16, 16)] = o16
        return carry
    lax.fori_loop(0, CHUNK // 16, fill_ones, 0)

    zbase = sid * ROWS_PER_TILE_ZERO
    base_row = (cid * NUM_SUBCORES + sid) * CHUNKS_PER_TILE

    for et, dI in enumerate((dI0, dI1, dI2, dI3)):
        pltpu.sync_copy(zdbuf, dacc.at[pl.ds(zbase, ROWS_PER_TILE_ZERO)])
        plsc.subcore_barrier()
        pltpu.sync_copy(dI.at[pl.ds(base_row, CHUNKS_PER_TILE)], dst_idx)

        def step(j, carry):
            pltpu.sync_copy(ones_v, dacc.at[dst_idx.at[j]], add=True)
            return carry
        lax.fori_loop(0, CHUNKS_PER_TILE, step, 0)
        plsc.subcore_barrier()
        pltpu.sync_copy(
            dacc.at[pl.ds(zbase, ROWS_PER_TILE_ZERO)],
            deg_out.at[pl.ds(et * NUM_CORES * ACC_ROWS + cid * ACC_ROWS
                             + zbase, ROWS_PER_TILE_ZERO)])


def _sc_deg(d0, d1, d2, d3):
    fn = pl.kernel(
        _sc_deg_body,
        mesh=plsc.VectorSubcoreMesh(core_axis_name="c", subcore_axis_name="s"),
        out_type=[
            jax.ShapeDtypeStruct((4 * NUM_CORES * ACC_ROWS,), jnp.float32),
        ],
        scratch_types=[
            pltpu.VMEM((CHUNKS_PER_TILE, CHUNK), jnp.int32),
            pltpu.VMEM((CHUNK,), jnp.float32),
            pltpu.VMEM((ROWS_PER_TILE_ZERO,), jnp.float32),
            pltpu.VMEM_SHARED((ACC_ROWS,), jnp.float32),
        ],
    )
    return fn(d0, d1, d2, d3)[0]


ROW_BLK = 1024
N_BLK = 10


def _tc_layer_body(pa, d0a, d1a, pb, d0b, d1b, p1, d01, d11, p2, d02, d12,
                   xc, xa, xb,
                   wla, wra, ba, wlb, wrb, bb, wl1, wl2, wrc, bc,
                   oc, oa, ob_ref):
    def mean(p, d0, d1):
        s = p[0] + p[1]
        deg = jnp.maximum(d0[...] + d1[...], 1.0)
        return s / deg[:, None]

    def lrelu(x):
        return jnp.where(x > 0, x, 0.01 * x)

    m_a = mean(pa[...], d0a, d1a)
    out_a = (jnp.dot(m_a, wla[...], preferred_element_type=jnp.float32)
             + jnp.dot(xa[...], wra[...], preferred_element_type=jnp.float32)
             + ba[...])
    oa[...] = lrelu(out_a)

    m_b = mean(pb[...], d0b, d1b)
    out_b = (jnp.dot(m_b, wlb[...], preferred_element_type=jnp.float32)
             + jnp.dot(xb[...], wrb[...], preferred_element_type=jnp.float32)
             + bb[...])
    ob_ref[...] = lrelu(out_b)

    m_1 = mean(p1[...], d01, d11)
    m_2 = mean(p2[...], d02, d12)
    out_c = (jnp.dot(m_1, wl1[...], preferred_element_type=jnp.float32)
             + jnp.dot(m_2, wl2[...], preferred_element_type=jnp.float32)
             + jnp.dot(xc[...], wrc[...], preferred_element_type=jnp.float32)
             + bc[...])
    oc[...] = lrelu(out_c)


def _tc_layer(pa, da, pb, db, p1, d1, p2, d2, xc, xa, xb,
              wla, wra, ba, wlb, wrb, bb, wl1, wl2, wrc, bc):
    p_spec = pl.BlockSpec((NUM_CORES, ROW_BLK, D_FEAT), lambda i: (0, i, 0))
    d_spec = pl.BlockSpec((ROW_BLK,), lambda i: (i,))
    x_spec = pl.BlockSpec((ROW_BLK, D_FEAT), lambda i: (i, 0))
    w_spec = pl.BlockSpec((D_FEAT, D_FEAT), lambda i: (0, 0))
    b_spec = pl.BlockSpec((1, D_FEAT), lambda i: (0, 0))
    degs = [da, db, d1, d2]
    return pl.pallas_call(
        _tc_layer_body,
        grid=(N_BLK,),
        in_specs=[p_spec, d_spec, d_spec, p_spec, d_spec, d_spec,
                  p_spec, d_spec, d_spec, p_spec, d_spec, d_spec,
                  x_spec, x_spec, x_spec,
                  w_spec, w_spec, b_spec, w_spec, w_spec, b_spec,
                  w_spec, w_spec, w_spec, b_spec],
        out_specs=[x_spec, x_spec, x_spec],
        out_shape=[jax.ShapeDtypeStruct((N_NODES, D_FEAT), jnp.float32)] * 3,
    )(pa, *degs[0], pb, *degs[1], p1, *degs[2], p2, *degs[3],
      xc, xa, xb,
      wla, wra, ba, wlb, wrb, bb, wl1, wl2, wrc, bc)


def _prep_edges(ei):
    pad = E_PAD - N_EDGES
    src = jnp.concatenate(
        [ei[0].astype(jnp.int32), jnp.zeros((pad,), jnp.int32)])
    dst = jnp.concatenate(
        [ei[1].astype(jnp.int32), jnp.full((pad,), N_NODES, jnp.int32)])
    return src.reshape(IDX_ROWS, CHUNK), dst.reshape(IDX_ROWS, CHUNK)


def kernel(x_cdr3b, x_tra_peptide, x_trb_peptide, edge_index_c2a,
           edge_index_c2b, edge_index_a2c, edge_index_b2c, params):
    xc, xa, xb = x_cdr3b, x_tra_peptide, x_trb_peptide
    e_c2a = _prep_edges(edge_index_c2a)
    e_c2b = _prep_edges(edge_index_c2b)
    e_a2c = _prep_edges(edge_index_a2c)
    e_b2c = _prep_edges(edge_index_b2c)

    deg_all = _sc_deg(e_c2a[1], e_c2b[1], e_a2c[1], e_b2c[1])
    degs = []
    for et in range(4):
        base = et * NUM_CORES * ACC_ROWS
        degs.append((deg_all[base:base + ACC_ROWS],
                     deg_all[base + ACC_ROWS:base + 2 * ACC_ROWS]))

    for lp in params:
        wla, ba, wra = lp["c2a"]
        wlb, bb, wrb = lp["c2b"]
        wl1, b1, wr1 = lp["a2c"]
        wl2, b2, wr2 = lp["b2c"]
        wrc = wr1 + wr2
        bc = (b1 + b2).reshape(1, D_FEAT)

        pa, pb, p1, p2 = _sc_layer(xc, xa, xb, e_c2a, e_c2b, e_a2c, e_b2c)

        xc, xa, xb = _tc_layer(
            pa, degs[0], pb, degs[1], p1, degs[2], p2, degs[3], xc, xa, xb,
            wla, wra, ba.reshape(1, D_FEAT),
            wlb, wrb, bb.reshape(1, D_FEAT),
            wl1, wl2, wrc, bc)

    return (xc, xa, xb)
